# baseline (speedup 1.0000x reference)
import functools

import jax
import jax.numpy as jnp
from jax import lax
from jax.experimental import pallas as pl
from jax.experimental.pallas import tpu as pltpu
from jax.experimental.pallas import tpu_sc as plsc

_P = 16
_NH = 24
_N = _NH * _NH
_HALF = _N // 2
_PC = 48
_ROWS_PER_IMG = 384 * _NH


def _mask_body(x_ref, m_ref):
    xb = x_ref[0]
    t = xb.reshape(_NH, _P, _NH, _PC)
    mx = jnp.max(jnp.max(t, axis=3), axis=1)
    m_ref[0] = (mx > 0.0).astype(jnp.int32).reshape(1, _N)


def _sc_body(masks_hbm, xrows_hbm, out_hbm, mv, fsrc, idx2, bufs, zbuf,
             gsem, wsem):
    wid = lax.axis_index("s") * 2 + lax.axis_index("c")
    b = wid // 2
    h = wid % 2
    iota = lax.broadcasted_iota(jnp.int32, (16,), 0)

    pltpu.sync_copy(masks_hbm.at[b], mv)

    def count_step(k, acc):
        m16f = mv[pl.ds(16 * k, 16)].astype(jnp.float32)
        return acc + jnp.sum(m16f)
    c_totf = lax.fori_loop(0, _N // 16, count_step, jnp.float32(0))
    c_tot = c_totf.astype(jnp.int32)

    def perm_step(k, kept):
        m16f = mv[pl.ds(16 * k, 16)].astype(jnp.float32)
        cs = plsc.cumsum(m16f)
        psum_ex = kept + cs - m16f
        n = 16 * k + iota
        nf = n.astype(jnp.float32)
        destf = jnp.where(m16f > 0, psum_ex, c_totf + nf - psum_ex)
        base = _ROWS_PER_IMG * b + 384 * (n // _NH) + (n % _NH)
        plsc.store_scatter(fsrc, [destf.astype(jnp.int32)], base)
        return kept + jnp.sum(m16f)
    lax.fori_loop(0, _N // 16, perm_step, jnp.float32(0))

    def idx_step(t, _):
        m = _HALF * h + t
        bb = plsc.load_gather(fsrc, [jnp.full((16,), 0, jnp.int32) + m])
        q = t // 8
        col = (t % 8) * 16
        idx2[q, pl.ds(col, 16)] = bb + _NH * iota
        return 0
    lax.fori_loop(0, _HALF, idx_step, 0)

    out_base = _ROWS_PER_IMG * b + 16 * _HALF * h
    def chunk_step(q, _):
        pltpu.async_copy(xrows_hbm.at[idx2.at[q]], bufs.at[0], gsem).wait()
        pltpu.sync_copy(bufs.at[0], out_hbm.at[pl.ds(out_base + 128 * q, 128)])
        return 0
    lax.fori_loop(0, 36, chunk_step, 0)

    for r in range(16):
        for t in range(3):
            zbuf[r, pl.ds(16 * t, 16)] = jnp.zeros((16,), jnp.float32)
    mz_lo = jnp.maximum(c_tot, _HALF * h)
    def zero_step(m, _):
        pltpu.sync_copy(
            zbuf, out_hbm.at[pl.ds(_ROWS_PER_IMG * b + 16 * m, 16)])
        return 0
    lax.fori_loop(mz_lo, _HALF * (h + 1), zero_step, 0)


def _sc_compact(masks, x_rows):
    nrows = x_rows.shape[0]
    mesh = plsc.VectorSubcoreMesh(
        core_axis_name="c", subcore_axis_name="s", num_cores=2,
        num_subcores=16)
    run = functools.partial(
        pl.kernel,
        out_type=jax.ShapeDtypeStruct((nrows, _PC), jnp.float32),
        mesh=mesh,
        scratch_types=[
            pltpu.VMEM((_N,), jnp.int32),
            pltpu.VMEM((_N,), jnp.int32),
            pltpu.VMEM((36, 128), jnp.int32),
            pltpu.VMEM((3, 128, _PC), jnp.float32),
            pltpu.VMEM((16, _PC), jnp.float32),
            pltpu.SemaphoreType.DMA,
            pltpu.SemaphoreType.DMA,
        ],
        compiler_params=pltpu.CompilerParams(
            needs_layout_passes=False, use_tc_tiling_on_sc=False),
    )(_sc_body)
    return run(masks, x_rows)


def kernel(images):
    B, H, W, C = images.shape
    x = images.reshape(B, H, W * C)
    masks3 = pl.pallas_call(
        _mask_body,
        grid=(B,),
        in_specs=[pl.BlockSpec((1, H, W * C), lambda i: (i, 0, 0))],
        out_specs=pl.BlockSpec((1, 1, _N), lambda i: (i, 0, 0)),
        out_shape=jax.ShapeDtypeStruct((B, 1, _N), jnp.int32),
    )(x)
    masks = masks3.reshape(B, _N)
    x_rows = images.reshape(B * _ROWS_PER_IMG, _PC)
    out_rows = _sc_compact(masks, x_rows)
    return out_rows.reshape(B, _N, _P, _P, C)


# speedup vs baseline: 26.5225x; 26.5225x over previous
import functools

import jax
import jax.numpy as jnp
from jax import lax
from jax.experimental import pallas as pl
from jax.experimental.pallas import tpu as pltpu
from jax.experimental.pallas import tpu_sc as plsc

_P = 16
_NH = 24
_N = _NH * _NH
_HALF = _N // 2
_PC = 48
_D = _P * _PC
_WC = _NH * _PC


def _mask_body(x_ref, m_ref):
    xb = x_ref[0]
    t = xb.reshape(_NH, _P, _NH, _PC)
    mx = jnp.max(jnp.max(t, axis=3), axis=1)
    m_ref[0] = (mx > 0.0).astype(jnp.int32).reshape(1, _N)


def _sc_body(masks_hbm, x_hbm, out_hbm, mv, destd, slab, staging, idxbuf,
             sem):
    s = lax.axis_index("s")
    cc_ax = lax.axis_index("c")
    b = s
    h = cc_ax
    iota = lax.broadcasted_iota(jnp.int32, (16,), 0)

    pltpu.sync_copy(masks_hbm.at[b], mv)

    def count_step(k, acc):
        tot, pre = acc
        m16f = mv[pl.ds(16 * k, 16)].astype(jnp.float32)
        sk = jnp.sum(m16f)
        pre = pre + jnp.where(k < 18 * h, sk, 0.0)
        return (tot + sk, pre)
    c_totf, kept_basef = lax.fori_loop(
        0, _N // 16, count_step, (jnp.float32(0), jnp.float32(0)))

    def perm_step(kk, kept):
        k = 18 * h + kk
        m16f = mv[pl.ds(16 * k, 16)].astype(jnp.float32)
        cs = plsc.cumsum(m16f)
        psum_ex = kept + cs - m16f
        n = 16 * k + iota
        nf = n.astype(jnp.float32)
        destf = jnp.where(m16f > 0, psum_ex, c_totf + nf - psum_ex)
        dro = _N * b + destf.astype(jnp.int32)
        sv = jnp.where(m16f > 0, dro, -dro - 1)
        nloc = n - _HALF * h
        pos = 32 * (nloc // _NH) + nloc % _NH
        plsc.store_scatter(destd, [pos], sv)
        return kept + jnp.sum(m16f)
    lax.fori_loop(0, _HALF // 16, perm_step, kept_basef)

    zeros16 = jnp.zeros((16,), jnp.float32)

    def slab_step(g, _):
        r = 12 * h + g
        pltpu.sync_copy(x_hbm.at[pl.ds(384 * b + _P * r, _P)], slab)
        dv1 = destd[pl.ds(32 * g, 16)]
        dv2 = destd[pl.ds(32 * g + 16, 16)]
        acc1 = iota * 0
        acc2 = iota * 0
        for cc in range(_NH):
            v = dv1[cc] if cc < 16 else dv2[cc - 16]
            kept = v >= 0
            dro = jnp.where(kept, v, -v - 1)
            if cc < 16:
                acc1 = jnp.where(iota == cc, dro, acc1)
            else:
                acc2 = jnp.where(iota == (cc - 16), dro, acc2)

            @pl.when(kept)
            def _assemble():
                for i in range(_P):
                    for t in range(3):
                        staging[cc, pl.ds(_PC * i + 16 * t, 16)] = (
                            slab[i, pl.ds(_PC * cc + 16 * t, 16)])

            @pl.when(jnp.logical_not(kept))
            def _zero():
                def zstep(tt, _z):
                    staging[cc, pl.ds(16 * tt, 16)] = zeros16
                    return 0
                lax.fori_loop(0, _D // 16, zstep, 0)

        idxbuf[pl.ds(0, 16)] = acc1
        idxbuf[pl.ds(16, 16)] = acc2
        pltpu.async_copy(staging, out_hbm.at[idxbuf.at[pl.ds(0, _NH)]],
                         sem).wait()
        return 0
    lax.fori_loop(0, _HALF // _NH, slab_step, 0)


def _sc_compact(masks, x):
    mesh = plsc.VectorSubcoreMesh(
        core_axis_name="c", subcore_axis_name="s", num_cores=2,
        num_subcores=16)
    run = functools.partial(
        pl.kernel,
        out_type=jax.ShapeDtypeStruct((masks.shape[0] * _N, _D), jnp.float32),
        mesh=mesh,
        scratch_types=[
            pltpu.VMEM((_N,), jnp.int32),
            pltpu.VMEM((2 * 384,), jnp.int32),
            pltpu.VMEM((_P, _WC), jnp.float32),
            pltpu.VMEM((_NH, _D), jnp.float32),
            pltpu.VMEM((32,), jnp.int32),
            pltpu.SemaphoreType.DMA,
        ],
        compiler_params=pltpu.CompilerParams(needs_layout_passes=False),
    )(_sc_body)
    return run(masks, x)


def kernel(images):
    B, H, W, C = images.shape
    x2 = images.reshape(B, H, W * C)
    masks3 = pl.pallas_call(
        _mask_body,
        grid=(B,),
        in_specs=[pl.BlockSpec((1, H, W * C), lambda i: (i, 0, 0))],
        out_specs=pl.BlockSpec((1, 1, _N), lambda i: (i, 0, 0)),
        out_shape=jax.ShapeDtypeStruct((B, 1, _N), jnp.int32),
    )(x2)
    masks = masks3.reshape(B, _N)
    x = images.reshape(B * H, W * C)
    out_rows = _sc_compact(masks, x)
    return out_rows.reshape(B, _N, _P, _P, C)
